# initial kernel scaffold (unmeasured)
import jax
import jax.numpy as jnp
from jax import lax
from jax.experimental import pallas as pl
from jax.experimental.pallas import tpu as pltpu


def kernel(
    x,
):
    def body(*refs):
        pass

    out_shape = jax.ShapeDtypeStruct(..., jnp.float32)
    return pl.pallas_call(body, out_shape=out_shape)(...)


# baseline (device time: 2345426 ns/iter reference)
import jax
import jax.numpy as jnp
from jax import lax
from jax.experimental import pallas as pl
from jax.experimental.pallas import tpu as pltpu

N_Z = 4
N_HOPS = N_Z - 1
M, N = 16384, 1024
BLK = 1024
N_BLK = M // BLK


def kernel(x):
    def body(x_ref, out_ref, comm_ref, send_sems, recv_sems):
        my_x = lax.axis_index("x")
        my_y = lax.axis_index("y")
        my_z = lax.axis_index("z")
        right = (my_z + 1) % N_Z
        left = (my_z - 1) % N_Z

        barrier = pltpu.get_barrier_semaphore()
        for nbr in (left, right):
            pl.semaphore_signal(
                barrier,
                inc=1,
                device_id=(my_x, my_y, nbr),
                device_id_type=pl.DeviceIdType.MESH,
            )
        pl.semaphore_wait(barrier, 2)

        acc = x_ref[...]
        for h in range(N_HOPS):
            rdma = pltpu.make_async_remote_copy(
                src_ref=x_ref if h == 0 else comm_ref.at[h - 1],
                dst_ref=comm_ref.at[h],
                send_sem=send_sems.at[h],
                recv_sem=recv_sems.at[h],
                device_id=(my_x, my_y, right),
                device_id_type=pl.DeviceIdType.MESH,
            )
            rdma.start()
            rdma.wait()
            acc = acc + comm_ref[h]
        out_ref[...] = acc

    return pl.pallas_call(
        body,
        grid=(N_BLK,),
        out_shape=jax.ShapeDtypeStruct((M, N), jnp.float32),
        in_specs=[pl.BlockSpec((BLK, N), lambda i: (i, 0))],
        out_specs=pl.BlockSpec((BLK, N), lambda i: (i, 0)),
        scratch_shapes=[
            pltpu.VMEM((N_HOPS, BLK, N), jnp.float32),
            pltpu.SemaphoreType.DMA((N_HOPS,)),
            pltpu.SemaphoreType.DMA((N_HOPS,)),
        ],
        compiler_params=pltpu.CompilerParams(
            collective_id=0,
            dimension_semantics=("arbitrary",),
            vmem_limit_bytes=64 * 1024 * 1024,
        ),
    )(x)


# device time: 1246359 ns/iter; 1.8818x vs baseline; 1.8818x over previous
import jax
import jax.numpy as jnp
from jax import lax
from jax.experimental import pallas as pl
from jax.experimental.pallas import tpu as pltpu

N_Z = 4
M, N = 16384, 1024
BLK = 2048
N_BLK = M // BLK
SEG = BLK // N_Z
N_STEPS = 2 * (N_Z - 1)


def kernel(x):
    def body(x_ref, out_ref, rs_recv, rs_send, ag_recv, send_sems, recv_sems):
        my_x = lax.axis_index("x")
        my_y = lax.axis_index("y")
        my_z = lax.axis_index("z")
        right = (my_z + 1) % N_Z
        left = (my_z - 1) % N_Z

        barrier = pltpu.get_barrier_semaphore()
        for nbr in (left, right):
            pl.semaphore_signal(
                barrier,
                inc=1,
                device_id=(my_x, my_y, nbr),
                device_id_type=pl.DeviceIdType.MESH,
            )
        pl.semaphore_wait(barrier, 2)

        def send(step, src, dst):
            rdma = pltpu.make_async_remote_copy(
                src_ref=src,
                dst_ref=dst,
                send_sem=send_sems.at[step],
                recv_sem=recv_sems.at[step],
                device_id=(my_x, my_y, right),
                device_id_type=pl.DeviceIdType.MESH,
            )
            rdma.start()
            rdma.wait()

        def xseg(j):
            return x_ref[pl.ds(j * SEG, SEG), :]

        send(0, x_ref.at[pl.ds(my_z * SEG, SEG), :], rs_recv.at[0])
        rs_send[0] = rs_recv[0] + xseg((my_z - 1) % N_Z)
        send(1, rs_send.at[0], rs_recv.at[1])
        rs_send[1] = rs_recv[1] + xseg((my_z - 2) % N_Z)
        send(2, rs_send.at[1], rs_recv.at[2])
        own = (my_z + 1) % N_Z
        red = rs_recv[2] + xseg(own)
        out_ref[pl.ds(own * SEG, SEG), :] = red
        rs_send[0] = red

        send(3, rs_send.at[0], ag_recv.at[0])
        out_ref[pl.ds(my_z * SEG, SEG), :] = ag_recv[0]
        send(4, ag_recv.at[0], ag_recv.at[1])
        out_ref[pl.ds(((my_z - 1) % N_Z) * SEG, SEG), :] = ag_recv[1]
        send(5, ag_recv.at[1], ag_recv.at[2])
        out_ref[pl.ds(((my_z - 2) % N_Z) * SEG, SEG), :] = ag_recv[2]

    return pl.pallas_call(
        body,
        grid=(N_BLK,),
        out_shape=jax.ShapeDtypeStruct((M, N), jnp.float32),
        in_specs=[pl.BlockSpec((BLK, N), lambda i: (i, 0))],
        out_specs=pl.BlockSpec((BLK, N), lambda i: (i, 0)),
        scratch_shapes=[
            pltpu.VMEM((3, SEG, N), jnp.float32),
            pltpu.VMEM((2, SEG, N), jnp.float32),
            pltpu.VMEM((3, SEG, N), jnp.float32),
            pltpu.SemaphoreType.DMA((N_STEPS,)),
            pltpu.SemaphoreType.DMA((N_STEPS,)),
        ],
        compiler_params=pltpu.CompilerParams(
            collective_id=0,
            dimension_semantics=("arbitrary",),
            vmem_limit_bytes=64 * 1024 * 1024,
        ),
    )(x)


# device time: 530058 ns/iter; 4.4248x vs baseline; 2.3514x over previous
import jax
import jax.numpy as jnp
from jax import lax
from jax.experimental import pallas as pl
from jax.experimental.pallas import tpu as pltpu

N_Z = 4
N_P = 8
M, N = 16384, 1024
BLK = M // N_P
ZSEG = BLK // N_Z
HALF = BLK // 2


def kernel(x):
    def body(
        x_ref,
        out_ref,
        x_block,
        red_block,
        rs_recv,
        rs_send,
        z_send_sems,
        z_recv_sems,
        cw_send_sems,
        cw_recv_sems,
        ccw_send_sems,
        ccw_recv_sems,
        local_sems,
    ):
        my_x = lax.axis_index("x")
        my_y = lax.axis_index("y")
        my_z = lax.axis_index("z")
        zright = (my_z + 1) % N_Z
        zleft = (my_z - 1) % N_Z

        p = jnp.where(my_x == 0, my_y, N_P - 1 - my_y)

        def ring_xy(q):
            q = q % N_P
            return jnp.where(q < 4, 0, 1), jnp.where(q < 4, q, N_P - 1 - q)

        nxt_x, nxt_y = ring_xy(p + 1)
        prv_x, prv_y = ring_xy(p - 1)

        barrier = pltpu.get_barrier_semaphore()
        for dev in (
            (my_x, my_y, zleft),
            (my_x, my_y, zright),
            (nxt_x, nxt_y, my_z),
            (prv_x, prv_y, my_z),
        ):
            pl.semaphore_signal(
                barrier, inc=1, device_id=dev,
                device_id_type=pl.DeviceIdType.MESH,
            )
        pl.semaphore_wait(barrier, 4)

        load = pltpu.make_async_copy(
            x_ref.at[pl.ds(p * BLK, BLK), :], x_block, local_sems.at[0]
        )
        load.start()
        load.wait()

        def zsend(step, src, dst):
            rdma = pltpu.make_async_remote_copy(
                src_ref=src,
                dst_ref=dst,
                send_sem=z_send_sems.at[step],
                recv_sem=z_recv_sems.at[step],
                device_id=(my_x, my_y, zright),
                device_id_type=pl.DeviceIdType.MESH,
            )
            rdma.start()
            rdma.wait()

        def xzseg(j):
            return x_block[pl.ds((j % N_Z) * ZSEG, ZSEG), :]

        zsend(0, x_block.at[pl.ds(my_z * ZSEG, ZSEG), :], rs_recv.at[0])
        rs_send[0] = rs_recv[0] + xzseg(my_z - 1)
        zsend(1, rs_send.at[0], rs_recv.at[1])
        rs_send[1] = rs_recv[1] + xzseg(my_z - 2)
        zsend(2, rs_send.at[1], rs_recv.at[2])
        own = (my_z + 1) % N_Z
        red_block[pl.ds(own * ZSEG, ZSEG), :] = rs_recv[2] + xzseg(own)

        for t in range(N_Z - 1):
            s = (own - t) % N_Z
            zsend(
                3 + t,
                red_block.at[pl.ds(s * ZSEG, ZSEG), :],
                red_block.at[pl.ds(s * ZSEG, ZSEG), :],
            )

        store = pltpu.make_async_copy(
            red_block, out_ref.at[pl.ds(p * BLK, BLK), :], local_sems.at[1]
        )
        store.start()

        for t in range(N_P - 1):
            qcw = (p - t) % N_P
            qcc = (p + t) % N_P
            if t == 0:
                cw_src = red_block.at[pl.ds(0, HALF), :]
                ccw_src = red_block.at[pl.ds(HALF, HALF), :]
            else:
                cw_src = out_ref.at[pl.ds(qcw * BLK, HALF), :]
                ccw_src = out_ref.at[pl.ds(qcc * BLK + HALF, HALF), :]
            cw = pltpu.make_async_remote_copy(
                src_ref=cw_src,
                dst_ref=out_ref.at[pl.ds(qcw * BLK, HALF), :],
                send_sem=cw_send_sems.at[t],
                recv_sem=cw_recv_sems.at[t],
                device_id=(nxt_x, nxt_y, my_z),
                device_id_type=pl.DeviceIdType.MESH,
            )
            ccw = pltpu.make_async_remote_copy(
                src_ref=ccw_src,
                dst_ref=out_ref.at[pl.ds(qcc * BLK + HALF, HALF), :],
                send_sem=ccw_send_sems.at[t],
                recv_sem=ccw_recv_sems.at[t],
                device_id=(prv_x, prv_y, my_z),
                device_id_type=pl.DeviceIdType.MESH,
            )
            cw.start()
            ccw.start()
            cw.wait()
            ccw.wait()

        store.wait()

    return pl.pallas_call(
        body,
        out_shape=jax.ShapeDtypeStruct((M, N), jnp.float32),
        in_specs=[pl.BlockSpec(memory_space=pl.ANY)],
        out_specs=pl.BlockSpec(memory_space=pl.ANY),
        scratch_shapes=[
            pltpu.VMEM((BLK, N), jnp.float32),
            pltpu.VMEM((BLK, N), jnp.float32),
            pltpu.VMEM((3, ZSEG, N), jnp.float32),
            pltpu.VMEM((2, ZSEG, N), jnp.float32),
            pltpu.SemaphoreType.DMA((6,)),
            pltpu.SemaphoreType.DMA((6,)),
            pltpu.SemaphoreType.DMA((N_P - 1,)),
            pltpu.SemaphoreType.DMA((N_P - 1,)),
            pltpu.SemaphoreType.DMA((N_P - 1,)),
            pltpu.SemaphoreType.DMA((N_P - 1,)),
            pltpu.SemaphoreType.DMA((2,)),
        ],
        compiler_params=pltpu.CompilerParams(
            collective_id=0,
            vmem_limit_bytes=64 * 1024 * 1024,
        ),
    )(x)
